# baseline (speedup 1.0000x reference)
import functools

import jax
import jax.numpy as jnp
from jax import lax
from jax.experimental import pallas as pl
from jax.experimental.pallas import tpu as pltpu
from jax.experimental.pallas import tpu_sc as plsc

EMBED_DIM = 64
B = 16384
MARGIN = 1.0
NC = 2
NS = 16
NW = NC * NS
ROWS_W = B // NW
CHUNK = 128
NCHUNK = ROWS_W // CHUNK
L = 16
GROUPS = CHUNK // L


def _vsqrt(s):
    s_safe = jnp.maximum(s, jnp.float32(1e-20))
    i = plsc.bitcast(s_safe, jnp.int32)
    i = jnp.int32(0x5F3759DF) - (i >> 1)
    y = plsc.bitcast(i, jnp.float32)
    half = s_safe * jnp.float32(0.5)
    for _ in range(3):
        y = y * (jnp.float32(1.5) - half * y * y)
    return s * y


_mesh = plsc.VectorSubcoreMesh(core_axis_name="c", subcore_axis_name="s")


@functools.partial(
    pl.kernel,
    mesh=_mesh,
    out_type=jax.ShapeDtypeStruct((NW, L), jnp.float32),
    compiler_params=pltpu.CompilerParams(
        needs_layout_passes=False, use_tc_tiling_on_sc=False
    ),
    scratch_types=[
        pltpu.VMEM((CHUNK,), jnp.int32),
        pltpu.VMEM((CHUNK,), jnp.int32),
        pltpu.VMEM((CHUNK,), jnp.int32),
        pltpu.VMEM((CHUNK,), jnp.int32),
        pltpu.VMEM((CHUNK,), jnp.int32),
        pltpu.VMEM((CHUNK, EMBED_DIM), jnp.float32),
        pltpu.VMEM((CHUNK, EMBED_DIM), jnp.float32),
        pltpu.VMEM((CHUNK, EMBED_DIM), jnp.float32),
        pltpu.VMEM((CHUNK, EMBED_DIM), jnp.float32),
        pltpu.VMEM((CHUNK, EMBED_DIM), jnp.float32),
        pltpu.VMEM((L,), jnp.float32),
        pltpu.SemaphoreType.DMA,
    ],
)
def _transe_kernel(ph_hbm, pt_hbm, nh_hbm, nt_hbm, r_hbm, e_hbm, re_hbm,
                   out_hbm,
                   phi_v, pti_v, nhi_v, nti_v, ri_v,
                   phr_v, ptr_v, nhr_v, ntr_v, rr_v,
                   loss_v, sem):
    wid = lax.axis_index("s") * NC + lax.axis_index("c")
    base = wid * ROWS_W
    lane = lax.iota(jnp.int32, L)

    def chunk_body(c, loss_acc):
        off = base + c * CHUNK
        pltpu.sync_copy(ph_hbm.at[pl.ds(off, CHUNK)], phi_v)
        pltpu.sync_copy(pt_hbm.at[pl.ds(off, CHUNK)], pti_v)
        pltpu.sync_copy(nh_hbm.at[pl.ds(off, CHUNK)], nhi_v)
        pltpu.sync_copy(nt_hbm.at[pl.ds(off, CHUNK)], nti_v)
        pltpu.sync_copy(r_hbm.at[pl.ds(off, CHUNK)], ri_v)
        c1 = pltpu.async_copy(e_hbm.at[phi_v], phr_v, sem)
        c2 = pltpu.async_copy(e_hbm.at[pti_v], ptr_v, sem)
        c3 = pltpu.async_copy(e_hbm.at[nhi_v], nhr_v, sem)
        c4 = pltpu.async_copy(e_hbm.at[nti_v], ntr_v, sem)
        c5 = pltpu.async_copy(re_hbm.at[ri_v], rr_v, sem)
        c1.wait()
        c2.wait()
        c3.wait()
        c4.wait()
        c5.wait()

        def group_body(g, acc):
            rows = g * L + lane

            def dim_body(j, carry):
                p_acc, n_acc = carry
                jv = jnp.full((L,), j, dtype=jnp.int32)
                ph = plsc.load_gather(phr_v, [rows, jv])
                pt = plsc.load_gather(ptr_v, [rows, jv])
                nh = plsc.load_gather(nhr_v, [rows, jv])
                nt = plsc.load_gather(ntr_v, [rows, jv])
                rv = plsc.load_gather(rr_v, [rows, jv])
                dp = ph + rv - pt
                dn = nh + rv - nt
                return (p_acc + dp * dp, n_acc + dn * dn)

            z = jnp.zeros((L,), jnp.float32)
            p_acc, n_acc = lax.fori_loop(0, EMBED_DIM, dim_body, (z, z))
            sp = _vsqrt(p_acc)
            sn = _vsqrt(n_acc)
            res = jnp.float32(MARGIN) + sp - sn
            return acc + jnp.maximum(res, jnp.float32(0.0))

        return lax.fori_loop(0, GROUPS, group_body, loss_acc)

    loss = lax.fori_loop(0, NCHUNK, chunk_body, jnp.zeros((L,), jnp.float32))
    loss_v[...] = loss
    pltpu.sync_copy(loss_v, out_hbm.at[wid])


def kernel(posi_head_list, posi_tail_list, nege_head_list, nege_tail_list,
           r_list, e_embed, r_embed):
    partials = _transe_kernel(posi_head_list, posi_tail_list,
                              nege_head_list, nege_tail_list,
                              r_list, e_embed, r_embed)
    return jnp.sum(partials)


# speedup vs baseline: 1.7857x; 1.7857x over previous
import functools

import jax
import jax.numpy as jnp
from jax import lax
from jax.experimental import pallas as pl
from jax.experimental.pallas import tpu as pltpu
from jax.experimental.pallas import tpu_sc as plsc

EMBED_DIM = 64
B = 16384
MARGIN = 1.0
NC = 2
NS = 16
NW = NC * NS
ROWS_W = B // NW
BATCH = 16
NBATCH = ROWS_W // BATCH
L = 16
NSEG = EMBED_DIM // L


def _vsqrt(s):
    s_safe = jnp.maximum(s, jnp.float32(1e-20))
    i = plsc.bitcast(s_safe, jnp.int32)
    i = jnp.int32(0x5F3759DF) - (i >> 1)
    y = plsc.bitcast(i, jnp.float32)
    half = s_safe * jnp.float32(0.5)
    for _ in range(3):
        y = y * (jnp.float32(1.5) - half * y * y)
    return s * y


def _lane_sum(v, lane):
    for step in (8, 4, 2, 1):
        perm = jnp.take_along_axis(
            v, lane ^ step, axis=0,
            mode=lax.GatherScatterMode.PROMISE_IN_BOUNDS,
        )
        v = v + perm
    return v


_mesh = plsc.VectorSubcoreMesh(core_axis_name="c", subcore_axis_name="s")


@functools.partial(
    pl.kernel,
    mesh=_mesh,
    out_type=jax.ShapeDtypeStruct((NW, L), jnp.float32),
    compiler_params=pltpu.CompilerParams(needs_layout_passes=False),
    scratch_types=[
        pltpu.VMEM((ROWS_W,), jnp.int32),
        pltpu.VMEM((ROWS_W,), jnp.int32),
        pltpu.VMEM((ROWS_W,), jnp.int32),
        pltpu.VMEM((ROWS_W,), jnp.int32),
        pltpu.VMEM((ROWS_W,), jnp.int32),
        pltpu.VMEM((2, BATCH, EMBED_DIM), jnp.float32),
        pltpu.VMEM((2, BATCH, EMBED_DIM), jnp.float32),
        pltpu.VMEM((2, BATCH, EMBED_DIM), jnp.float32),
        pltpu.VMEM((2, BATCH, EMBED_DIM), jnp.float32),
        pltpu.VMEM((2, BATCH, EMBED_DIM), jnp.float32),
        pltpu.VMEM((L,), jnp.float32),
        pltpu.SemaphoreType.DMA,
    ],
)
def _transe_kernel(ph_hbm, pt_hbm, nh_hbm, nt_hbm, r_hbm, e_hbm, re_hbm,
                   out_hbm,
                   phi_v, pti_v, nhi_v, nti_v, ri_v,
                   phb, ptb, nhb, ntb, rb,
                   loss_v, sem):
    wid = lax.axis_index("s") * NC + lax.axis_index("c")
    base = wid * ROWS_W
    lane = lax.iota(jnp.int32, L)

    pltpu.sync_copy(ph_hbm.at[pl.ds(base, ROWS_W)], phi_v)
    pltpu.sync_copy(pt_hbm.at[pl.ds(base, ROWS_W)], pti_v)
    pltpu.sync_copy(nh_hbm.at[pl.ds(base, ROWS_W)], nhi_v)
    pltpu.sync_copy(nt_hbm.at[pl.ds(base, ROWS_W)], nti_v)
    pltpu.sync_copy(r_hbm.at[pl.ds(base, ROWS_W)], ri_v)

    def fire(mb, slot):
        off = mb * BATCH
        phv = phi_v[pl.ds(off, BATCH)]
        ptv = pti_v[pl.ds(off, BATCH)]
        nhv = nhi_v[pl.ds(off, BATCH)]
        ntv = nti_v[pl.ds(off, BATCH)]
        rv = ri_v[pl.ds(off, BATCH)]
        for k in range(BATCH):
            pltpu.async_copy(e_hbm.at[phv[k]], phb.at[slot, k], sem)
            pltpu.async_copy(e_hbm.at[ptv[k]], ptb.at[slot, k], sem)
            pltpu.async_copy(e_hbm.at[nhv[k]], nhb.at[slot, k], sem)
            pltpu.async_copy(e_hbm.at[ntv[k]], ntb.at[slot, k], sem)
            pltpu.async_copy(re_hbm.at[rv[k]], rb.at[slot, k], sem)

    def drain(slot):
        pltpu.make_async_copy(e_hbm.at[pl.ds(0, BATCH)], phb.at[slot], sem).wait()
        pltpu.make_async_copy(e_hbm.at[pl.ds(0, BATCH)], ptb.at[slot], sem).wait()
        pltpu.make_async_copy(e_hbm.at[pl.ds(0, BATCH)], nhb.at[slot], sem).wait()
        pltpu.make_async_copy(e_hbm.at[pl.ds(0, BATCH)], ntb.at[slot], sem).wait()
        pltpu.make_async_copy(re_hbm.at[pl.ds(0, BATCH)], rb.at[slot], sem).wait()

    def compute(slot, loss_acc):
        zero = jnp.zeros((L,), jnp.float32)
        pa_vec = zero
        na_vec = zero
        for r in range(BATCH):
            pacc = zero
            nacc = zero
            for s in range(NSEG):
                sl = pl.ds(s * L, L)
                ph = phb[slot, r, sl]
                pt = ptb[slot, r, sl]
                nh = nhb[slot, r, sl]
                nt = ntb[slot, r, sl]
                rr = rb[slot, r, sl]
                dp = ph + rr - pt
                dn = nh + rr - nt
                pacc = pacc + dp * dp
                nacc = nacc + dn * dn
            psum = _lane_sum(pacc, lane)
            nsum = _lane_sum(nacc, lane)
            pa_vec = jnp.where(lane == r, psum, pa_vec)
            na_vec = jnp.where(lane == r, nsum, na_vec)
        sp = _vsqrt(pa_vec)
        sn = _vsqrt(na_vec)
        res = jnp.float32(MARGIN) + sp - sn
        return loss_acc + jnp.maximum(res, jnp.float32(0.0))

    def fire_guarded(mb, slot):
        @pl.when(mb < NBATCH)
        def _():
            fire(mb, slot)

    fire(jnp.int32(0), 0)

    def pair_body(mb, loss_acc):
        fire_guarded(mb + 1, 1)
        drain(0)
        loss_acc = compute(0, loss_acc)
        fire_guarded(mb + 2, 0)
        drain(1)
        loss_acc = compute(1, loss_acc)
        return loss_acc

    loss = lax.fori_loop(
        0, NBATCH // 2,
        lambda i, acc: pair_body(i * 2, acc),
        jnp.zeros((L,), jnp.float32),
    )
    loss_v[...] = loss
    pltpu.sync_copy(loss_v, out_hbm.at[wid])


def kernel(posi_head_list, posi_tail_list, nege_head_list, nege_tail_list,
           r_list, e_embed, r_embed):
    partials = _transe_kernel(posi_head_list, posi_tail_list,
                              nege_head_list, nege_tail_list,
                              r_list, e_embed, r_embed)
    return jnp.sum(partials)


# speedup vs baseline: 2.1498x; 1.2039x over previous
import functools

import jax
import jax.numpy as jnp
from jax import lax
from jax.experimental import pallas as pl
from jax.experimental.pallas import tpu as pltpu
from jax.experimental.pallas import tpu_sc as plsc

EMBED_DIM = 64
E_NUM = 1000000
B = 16384
MARGIN = 1.0
NC = 2
NS = 16
NW = NC * NS
ROWS_W = B // NW
BATCH = 16
NBATCH = ROWS_W // BATCH
L = 16
NSEG = EMBED_DIM // L

TBLK = 8192
TGRID = -(-E_NUM // TBLK)


def _tt_body(in_ref, out_ref):
    blk = in_ref[...]
    i = lax.broadcasted_iota(jnp.int32, (EMBED_DIM, EMBED_DIM), 0)
    j = lax.broadcasted_iota(jnp.int32, (EMBED_DIM, EMBED_DIM), 1)
    eye = (i == j).astype(jnp.float32)
    out_ref[...] = lax.dot_general(
        blk, eye, (((0,), (0,)), ((), ())),
        preferred_element_type=jnp.float32,
    )


_tc_transpose = pl.pallas_call(
    _tt_body,
    grid=(TGRID,),
    in_specs=[pl.BlockSpec((EMBED_DIM, TBLK), lambda i: (0, i))],
    out_specs=pl.BlockSpec((TBLK, EMBED_DIM), lambda i: (i, 0)),
    out_shape=jax.ShapeDtypeStruct((E_NUM, EMBED_DIM), jnp.float32),
)


def _vsqrt(s):
    s_safe = jnp.maximum(s, jnp.float32(1e-20))
    i = plsc.bitcast(s_safe, jnp.int32)
    i = jnp.int32(0x5F3759DF) - (i >> 1)
    y = plsc.bitcast(i, jnp.float32)
    half = s_safe * jnp.float32(0.5)
    for _ in range(3):
        y = y * (jnp.float32(1.5) - half * y * y)
    return s * y


def _lane_sum(v, lane):
    for step in (8, 4, 2, 1):
        perm = jnp.take_along_axis(
            v, lane ^ step, axis=0,
            mode=lax.GatherScatterMode.PROMISE_IN_BOUNDS,
        )
        v = v + perm
    return v


_mesh = plsc.VectorSubcoreMesh(core_axis_name="c", subcore_axis_name="s")


@functools.partial(
    pl.kernel,
    mesh=_mesh,
    out_type=jax.ShapeDtypeStruct((NW, L), jnp.float32),
    compiler_params=pltpu.CompilerParams(needs_layout_passes=False),
    scratch_types=[
        pltpu.VMEM((ROWS_W,), jnp.int32),
        pltpu.VMEM((ROWS_W,), jnp.int32),
        pltpu.VMEM((ROWS_W,), jnp.int32),
        pltpu.VMEM((ROWS_W,), jnp.int32),
        pltpu.VMEM((ROWS_W,), jnp.int32),
        pltpu.VMEM((2, BATCH, EMBED_DIM), jnp.float32),
        pltpu.VMEM((2, BATCH, EMBED_DIM), jnp.float32),
        pltpu.VMEM((2, BATCH, EMBED_DIM), jnp.float32),
        pltpu.VMEM((2, BATCH, EMBED_DIM), jnp.float32),
        pltpu.VMEM((2, BATCH, EMBED_DIM), jnp.float32),
        pltpu.VMEM((L,), jnp.float32),
        pltpu.SemaphoreType.DMA,
    ],
)
def _transe_kernel(ph_hbm, pt_hbm, nh_hbm, nt_hbm, r_hbm, e_hbm, re_hbm,
                   out_hbm,
                   phi_v, pti_v, nhi_v, nti_v, ri_v,
                   phb, ptb, nhb, ntb, rb,
                   loss_v, sem):
    wid = lax.axis_index("s") * NC + lax.axis_index("c")
    base = wid * ROWS_W
    lane = lax.iota(jnp.int32, L)

    pltpu.sync_copy(ph_hbm.at[pl.ds(base, ROWS_W)], phi_v)
    pltpu.sync_copy(pt_hbm.at[pl.ds(base, ROWS_W)], pti_v)
    pltpu.sync_copy(nh_hbm.at[pl.ds(base, ROWS_W)], nhi_v)
    pltpu.sync_copy(nt_hbm.at[pl.ds(base, ROWS_W)], nti_v)
    pltpu.sync_copy(r_hbm.at[pl.ds(base, ROWS_W)], ri_v)

    def fire(mb, slot):
        off = mb * BATCH
        phv = phi_v[pl.ds(off, BATCH)]
        ptv = pti_v[pl.ds(off, BATCH)]
        nhv = nhi_v[pl.ds(off, BATCH)]
        ntv = nti_v[pl.ds(off, BATCH)]
        rv = ri_v[pl.ds(off, BATCH)]
        for k in range(BATCH):
            pltpu.async_copy(e_hbm.at[phv[k]], phb.at[slot, k], sem)
            pltpu.async_copy(e_hbm.at[ptv[k]], ptb.at[slot, k], sem)
            pltpu.async_copy(e_hbm.at[nhv[k]], nhb.at[slot, k], sem)
            pltpu.async_copy(e_hbm.at[ntv[k]], ntb.at[slot, k], sem)
            pltpu.async_copy(re_hbm.at[rv[k]], rb.at[slot, k], sem)

    def drain(slot):
        pltpu.make_async_copy(e_hbm.at[pl.ds(0, BATCH)], phb.at[slot], sem).wait()
        pltpu.make_async_copy(e_hbm.at[pl.ds(0, BATCH)], ptb.at[slot], sem).wait()
        pltpu.make_async_copy(e_hbm.at[pl.ds(0, BATCH)], nhb.at[slot], sem).wait()
        pltpu.make_async_copy(e_hbm.at[pl.ds(0, BATCH)], ntb.at[slot], sem).wait()
        pltpu.make_async_copy(re_hbm.at[pl.ds(0, BATCH)], rb.at[slot], sem).wait()

    def compute(slot, loss_acc):
        zero = jnp.zeros((L,), jnp.float32)
        pa_vec = zero
        na_vec = zero
        for r in range(BATCH):
            pacc = zero
            nacc = zero
            for s in range(NSEG):
                sl = pl.ds(s * L, L)
                ph = phb[slot, r, sl]
                pt = ptb[slot, r, sl]
                nh = nhb[slot, r, sl]
                nt = ntb[slot, r, sl]
                rr = rb[slot, r, sl]
                dp = ph + rr - pt
                dn = nh + rr - nt
                pacc = pacc + dp * dp
                nacc = nacc + dn * dn
            psum = _lane_sum(pacc, lane)
            nsum = _lane_sum(nacc, lane)
            pa_vec = jnp.where(lane == r, psum, pa_vec)
            na_vec = jnp.where(lane == r, nsum, na_vec)
        sp = _vsqrt(pa_vec)
        sn = _vsqrt(na_vec)
        res = jnp.float32(MARGIN) + sp - sn
        return loss_acc + jnp.maximum(res, jnp.float32(0.0))

    def fire_guarded(mb, slot):
        @pl.when(mb < NBATCH)
        def _():
            fire(mb, slot)

    fire(jnp.int32(0), 0)

    def pair_body(mb, loss_acc):
        fire_guarded(mb + 1, 1)
        drain(0)
        loss_acc = compute(0, loss_acc)
        fire_guarded(mb + 2, 0)
        drain(1)
        loss_acc = compute(1, loss_acc)
        return loss_acc

    loss = lax.fori_loop(
        0, NBATCH // 2,
        lambda i, acc: pair_body(i * 2, acc),
        jnp.zeros((L,), jnp.float32),
    )
    loss_v[...] = loss
    pltpu.sync_copy(loss_v, out_hbm.at[wid])


def kernel(posi_head_list, posi_tail_list, nege_head_list, nege_tail_list,
           r_list, e_embed, r_embed):
    e_rowmajor = _tc_transpose(e_embed.T)
    partials = _transe_kernel(posi_head_list, posi_tail_list,
                              nege_head_list, nege_tail_list,
                              r_list, e_rowmajor, r_embed)
    return jnp.sum(partials)


# speedup vs baseline: 2.3163x; 1.0774x over previous
import functools

import jax
import jax.numpy as jnp
from jax import lax
from jax.experimental import pallas as pl
from jax.experimental.pallas import tpu as pltpu
from jax.experimental.pallas import tpu_sc as plsc

EMBED_DIM = 64
E_NUM = 1000000
B = 16384
MARGIN = 1.0
NC = 2
NS = 16
NW = NC * NS
ROWS_W = B // NW
BATCH = 16
NBATCH = ROWS_W // BATCH
L = 16
NSEG = EMBED_DIM // L

TBLK = 16384
TGRID = -(-E_NUM // TBLK)


def _tt_body(in_ref, out_ref):
    blk = in_ref[...]
    i = lax.broadcasted_iota(jnp.int32, (EMBED_DIM, EMBED_DIM), 0)
    j = lax.broadcasted_iota(jnp.int32, (EMBED_DIM, EMBED_DIM), 1)
    eye = (i == j).astype(jnp.float32)
    out_ref[...] = lax.dot_general(
        blk, eye, (((0,), (0,)), ((), ())),
        preferred_element_type=jnp.float32,
    )


_tc_transpose = pl.pallas_call(
    _tt_body,
    grid=(TGRID,),
    in_specs=[pl.BlockSpec((EMBED_DIM, TBLK), lambda i: (0, i))],
    out_specs=pl.BlockSpec((TBLK, EMBED_DIM), lambda i: (i, 0)),
    out_shape=jax.ShapeDtypeStruct((E_NUM, EMBED_DIM), jnp.float32),
)


def _vsqrt(s):
    s_safe = jnp.maximum(s, jnp.float32(1e-20))
    i = plsc.bitcast(s_safe, jnp.int32)
    i = jnp.int32(0x5F3759DF) - (i >> 1)
    y = plsc.bitcast(i, jnp.float32)
    half = s_safe * jnp.float32(0.5)
    for _ in range(3):
        y = y * (jnp.float32(1.5) - half * y * y)
    return s * y


def _lane_sum(v, lane):
    for step in (8, 4, 2, 1):
        perm = jnp.take_along_axis(
            v, lane ^ step, axis=0,
            mode=lax.GatherScatterMode.PROMISE_IN_BOUNDS,
        )
        v = v + perm
    return v


_mesh = plsc.VectorSubcoreMesh(core_axis_name="c", subcore_axis_name="s")


@functools.partial(
    pl.kernel,
    mesh=_mesh,
    out_type=jax.ShapeDtypeStruct((NW, L), jnp.float32),
    compiler_params=pltpu.CompilerParams(needs_layout_passes=False),
    scratch_types=[
        pltpu.VMEM((ROWS_W,), jnp.int32),
        pltpu.VMEM((ROWS_W,), jnp.int32),
        pltpu.VMEM((ROWS_W,), jnp.int32),
        pltpu.VMEM((ROWS_W,), jnp.int32),
        pltpu.VMEM((ROWS_W,), jnp.int32),
        pltpu.VMEM((2, BATCH, EMBED_DIM), jnp.float32),
        pltpu.VMEM((2, BATCH, EMBED_DIM), jnp.float32),
        pltpu.VMEM((2, BATCH, EMBED_DIM), jnp.float32),
        pltpu.VMEM((2, BATCH, EMBED_DIM), jnp.float32),
        pltpu.VMEM((2, BATCH, EMBED_DIM), jnp.float32),
        pltpu.VMEM((L,), jnp.float32),
        pltpu.SemaphoreType.DMA,
    ],
)
def _transe_kernel(ph_hbm, pt_hbm, nh_hbm, nt_hbm, r_hbm, e_hbm, re_hbm,
                   out_hbm,
                   phi_v, pti_v, nhi_v, nti_v, ri_v,
                   phb, ptb, nhb, ntb, rb,
                   loss_v, sem):
    wid = lax.axis_index("s") * NC + lax.axis_index("c")
    base = wid * ROWS_W
    lane = lax.iota(jnp.int32, L)

    pltpu.sync_copy(ph_hbm.at[pl.ds(base, ROWS_W)], phi_v)
    pltpu.sync_copy(pt_hbm.at[pl.ds(base, ROWS_W)], pti_v)
    pltpu.sync_copy(nh_hbm.at[pl.ds(base, ROWS_W)], nhi_v)
    pltpu.sync_copy(nt_hbm.at[pl.ds(base, ROWS_W)], nti_v)
    pltpu.sync_copy(r_hbm.at[pl.ds(base, ROWS_W)], ri_v)

    def fire(mb, slot):
        off = mb * BATCH
        phv = phi_v[pl.ds(off, BATCH)]
        ptv = pti_v[pl.ds(off, BATCH)]
        nhv = nhi_v[pl.ds(off, BATCH)]
        ntv = nti_v[pl.ds(off, BATCH)]
        rv = ri_v[pl.ds(off, BATCH)]
        for k in range(BATCH):
            pltpu.async_copy(e_hbm.at[phv[k]], phb.at[slot, k], sem)
            pltpu.async_copy(e_hbm.at[ptv[k]], ptb.at[slot, k], sem)
            pltpu.async_copy(e_hbm.at[nhv[k]], nhb.at[slot, k], sem)
            pltpu.async_copy(e_hbm.at[ntv[k]], ntb.at[slot, k], sem)
            pltpu.async_copy(re_hbm.at[rv[k]], rb.at[slot, k], sem)

    def drain(slot):
        pltpu.make_async_copy(e_hbm.at[pl.ds(0, BATCH)], phb.at[slot], sem).wait()
        pltpu.make_async_copy(e_hbm.at[pl.ds(0, BATCH)], ptb.at[slot], sem).wait()
        pltpu.make_async_copy(e_hbm.at[pl.ds(0, BATCH)], nhb.at[slot], sem).wait()
        pltpu.make_async_copy(e_hbm.at[pl.ds(0, BATCH)], ntb.at[slot], sem).wait()
        pltpu.make_async_copy(re_hbm.at[pl.ds(0, BATCH)], rb.at[slot], sem).wait()

    def compute(slot, loss_acc):
        zero = jnp.zeros((L,), jnp.float32)
        pa_vec = zero
        na_vec = zero
        for r in range(BATCH):
            pacc = zero
            nacc = zero
            for s in range(NSEG):
                sl = pl.ds(s * L, L)
                ph = phb[slot, r, sl]
                pt = ptb[slot, r, sl]
                nh = nhb[slot, r, sl]
                nt = ntb[slot, r, sl]
                rr = rb[slot, r, sl]
                dp = ph + rr - pt
                dn = nh + rr - nt
                pacc = pacc + dp * dp
                nacc = nacc + dn * dn
            psum = _lane_sum(pacc, lane)
            nsum = _lane_sum(nacc, lane)
            pa_vec = jnp.where(lane == r, psum, pa_vec)
            na_vec = jnp.where(lane == r, nsum, na_vec)
        sp = _vsqrt(pa_vec)
        sn = _vsqrt(na_vec)
        res = jnp.float32(MARGIN) + sp - sn
        return loss_acc + jnp.maximum(res, jnp.float32(0.0))

    def fire_guarded(mb, slot):
        @pl.when(mb < NBATCH)
        def _():
            fire(mb, slot)

    fire(jnp.int32(0), 0)

    def pair_body(mb, loss_acc):
        fire_guarded(mb + 1, 1)
        drain(0)
        loss_acc = compute(0, loss_acc)
        fire_guarded(mb + 2, 0)
        drain(1)
        loss_acc = compute(1, loss_acc)
        return loss_acc

    loss = lax.fori_loop(
        0, NBATCH // 2,
        lambda i, acc: pair_body(i * 2, acc),
        jnp.zeros((L,), jnp.float32),
    )
    loss_v[...] = loss
    pltpu.sync_copy(loss_v, out_hbm.at[wid])


def kernel(posi_head_list, posi_tail_list, nege_head_list, nege_tail_list,
           r_list, e_embed, r_embed):
    e_rowmajor = _tc_transpose(e_embed.T)
    partials = _transe_kernel(posi_head_list, posi_tail_list,
                              nege_head_list, nege_tail_list,
                              r_list, e_rowmajor, r_embed)
    return jnp.sum(partials)


# speedup vs baseline: 2.3343x; 1.0078x over previous
import functools

import jax
import jax.numpy as jnp
from jax import lax
from jax.experimental import pallas as pl
from jax.experimental.pallas import tpu as pltpu
from jax.experimental.pallas import tpu_sc as plsc

EMBED_DIM = 64
E_NUM = 1000000
B = 16384
MARGIN = 1.0
NC = 2
NS = 16
NW = NC * NS
ROWS_W = B // NW
BATCH = 16
NBATCH = ROWS_W // BATCH
L = 16
NSEG = EMBED_DIM // L

TBLK = 16384
TGRID = -(-E_NUM // TBLK)


def _tt_body(in_ref, out_ref):
    out_ref[...] = in_ref[...].T


_tc_transpose = pl.pallas_call(
    _tt_body,
    grid=(TGRID,),
    in_specs=[pl.BlockSpec((EMBED_DIM, TBLK), lambda i: (0, i))],
    out_specs=pl.BlockSpec((TBLK, EMBED_DIM), lambda i: (i, 0)),
    out_shape=jax.ShapeDtypeStruct((E_NUM, EMBED_DIM), jnp.float32),
)


def _vsqrt(s):
    s_safe = jnp.maximum(s, jnp.float32(1e-20))
    i = plsc.bitcast(s_safe, jnp.int32)
    i = jnp.int32(0x5F3759DF) - (i >> 1)
    y = plsc.bitcast(i, jnp.float32)
    half = s_safe * jnp.float32(0.5)
    for _ in range(3):
        y = y * (jnp.float32(1.5) - half * y * y)
    return s * y


def _lane_sum(v, lane):
    for step in (8, 4, 2, 1):
        perm = jnp.take_along_axis(
            v, lane ^ step, axis=0,
            mode=lax.GatherScatterMode.PROMISE_IN_BOUNDS,
        )
        v = v + perm
    return v


_mesh = plsc.VectorSubcoreMesh(core_axis_name="c", subcore_axis_name="s")


@functools.partial(
    pl.kernel,
    mesh=_mesh,
    out_type=jax.ShapeDtypeStruct((NW, L), jnp.float32),
    compiler_params=pltpu.CompilerParams(needs_layout_passes=False),
    scratch_types=[
        pltpu.VMEM((ROWS_W,), jnp.int32),
        pltpu.VMEM((ROWS_W,), jnp.int32),
        pltpu.VMEM((ROWS_W,), jnp.int32),
        pltpu.VMEM((ROWS_W,), jnp.int32),
        pltpu.VMEM((ROWS_W,), jnp.int32),
        pltpu.VMEM((2, BATCH, EMBED_DIM), jnp.float32),
        pltpu.VMEM((2, BATCH, EMBED_DIM), jnp.float32),
        pltpu.VMEM((2, BATCH, EMBED_DIM), jnp.float32),
        pltpu.VMEM((2, BATCH, EMBED_DIM), jnp.float32),
        pltpu.VMEM((2, BATCH, EMBED_DIM), jnp.float32),
        pltpu.VMEM((L,), jnp.float32),
        pltpu.SemaphoreType.DMA,
    ],
)
def _transe_kernel(ph_hbm, pt_hbm, nh_hbm, nt_hbm, r_hbm, e_hbm, re_hbm,
                   out_hbm,
                   phi_v, pti_v, nhi_v, nti_v, ri_v,
                   phb, ptb, nhb, ntb, rb,
                   loss_v, sem):
    wid = lax.axis_index("s") * NC + lax.axis_index("c")
    base = wid * ROWS_W
    lane = lax.iota(jnp.int32, L)

    pltpu.sync_copy(ph_hbm.at[pl.ds(base, ROWS_W)], phi_v)
    pltpu.sync_copy(pt_hbm.at[pl.ds(base, ROWS_W)], pti_v)
    pltpu.sync_copy(nh_hbm.at[pl.ds(base, ROWS_W)], nhi_v)
    pltpu.sync_copy(nt_hbm.at[pl.ds(base, ROWS_W)], nti_v)
    pltpu.sync_copy(r_hbm.at[pl.ds(base, ROWS_W)], ri_v)

    def fire(mb, slot):
        off = mb * BATCH
        phv = phi_v[pl.ds(off, BATCH)]
        ptv = pti_v[pl.ds(off, BATCH)]
        nhv = nhi_v[pl.ds(off, BATCH)]
        ntv = nti_v[pl.ds(off, BATCH)]
        rv = ri_v[pl.ds(off, BATCH)]
        for k in range(BATCH):
            pltpu.async_copy(e_hbm.at[phv[k]], phb.at[slot, k], sem)
            pltpu.async_copy(e_hbm.at[ptv[k]], ptb.at[slot, k], sem)
            pltpu.async_copy(e_hbm.at[nhv[k]], nhb.at[slot, k], sem)
            pltpu.async_copy(e_hbm.at[ntv[k]], ntb.at[slot, k], sem)
            pltpu.async_copy(re_hbm.at[rv[k]], rb.at[slot, k], sem)

    def drain(slot):
        pltpu.make_async_copy(e_hbm.at[pl.ds(0, BATCH)], phb.at[slot], sem).wait()
        pltpu.make_async_copy(e_hbm.at[pl.ds(0, BATCH)], ptb.at[slot], sem).wait()
        pltpu.make_async_copy(e_hbm.at[pl.ds(0, BATCH)], nhb.at[slot], sem).wait()
        pltpu.make_async_copy(e_hbm.at[pl.ds(0, BATCH)], ntb.at[slot], sem).wait()
        pltpu.make_async_copy(re_hbm.at[pl.ds(0, BATCH)], rb.at[slot], sem).wait()

    def compute(slot, loss_acc):
        zero = jnp.zeros((L,), jnp.float32)
        pa_vec = zero
        na_vec = zero
        for r in range(BATCH):
            pacc = zero
            nacc = zero
            for s in range(NSEG):
                sl = pl.ds(s * L, L)
                ph = phb[slot, r, sl]
                pt = ptb[slot, r, sl]
                nh = nhb[slot, r, sl]
                nt = ntb[slot, r, sl]
                rr = rb[slot, r, sl]
                dp = ph + rr - pt
                dn = nh + rr - nt
                pacc = pacc + dp * dp
                nacc = nacc + dn * dn
            psum = _lane_sum(pacc, lane)
            nsum = _lane_sum(nacc, lane)
            pa_vec = jnp.where(lane == r, psum, pa_vec)
            na_vec = jnp.where(lane == r, nsum, na_vec)
        sp = _vsqrt(pa_vec)
        sn = _vsqrt(na_vec)
        res = jnp.float32(MARGIN) + sp - sn
        return loss_acc + jnp.maximum(res, jnp.float32(0.0))

    def fire_guarded(mb, slot):
        @pl.when(mb < NBATCH)
        def _():
            fire(mb, slot)

    fire(jnp.int32(0), 0)

    def pair_body(mb, loss_acc):
        fire_guarded(mb + 1, 1)
        drain(0)
        loss_acc = compute(0, loss_acc)
        fire_guarded(mb + 2, 0)
        drain(1)
        loss_acc = compute(1, loss_acc)
        return loss_acc

    loss = lax.fori_loop(
        0, NBATCH // 2,
        lambda i, acc: pair_body(i * 2, acc),
        jnp.zeros((L,), jnp.float32),
    )
    loss_v[...] = loss
    pltpu.sync_copy(loss_v, out_hbm.at[wid])


def kernel(posi_head_list, posi_tail_list, nege_head_list, nege_tail_list,
           r_list, e_embed, r_embed):
    e_rowmajor = _tc_transpose(e_embed.T)
    partials = _transe_kernel(posi_head_list, posi_tail_list,
                              nege_head_list, nege_tail_list,
                              r_list, e_rowmajor, r_embed)
    return jnp.sum(partials)
